# baseline (speedup 1.0000x reference)
import functools

import jax
import jax.numpy as jnp
from jax.experimental import pallas as pl

_BS = 16


def _log_sigmoid(x):
    return jnp.minimum(x, 0.0) - jnp.log1p(jnp.exp(-jnp.abs(x)))


def _focal(logp):
    p = jnp.exp(logp)
    return -0.25 * (1.0 - p) ** 2 * logp


def _batch_part(logits, boxes_t, cut_row, tboxes_t, tlabels2, tcut2,
                src2, tgt2, *, B, Q, C1, N):
    num_classes = C1 - 1
    exp_x = jnp.exp(logits).astype(jnp.bfloat16)
    ones_c = jnp.ones((C1, 1), jnp.bfloat16)
    s = jnp.dot(exp_x, ones_c, preferred_element_type=jnp.float32)

    x255 = logits[:, num_classes:C1]
    ce0_sum = jnp.sum(_focal(x255 - jnp.log(s)))

    tgt_ohT = (jax.lax.broadcasted_iota(jnp.int32, (N, N), 0)
               == tgt2).astype(jnp.float32)
    tgt_cut_m = jnp.dot(tcut2, tgt_ohT,
                        preferred_element_type=jnp.float32)
    tgt_boxes_t = jnp.dot(tboxes_t, tgt_ohT,
                          preferred_element_type=jnp.float32)

    matchf = (jax.lax.broadcasted_iota(jnp.int32, (Q, N), 0)
              == src2).astype(jnp.float32)

    src_col = src2.reshape(N, 1)
    tgt_col = tgt2.reshape(N, 1)
    src_oh = (jax.lax.broadcasted_iota(jnp.int32, (N, Q), 1)
              == src_col).astype(jnp.bfloat16)
    rows_m = jnp.dot(src_oh, logits.astype(jnp.bfloat16),
                     preferred_element_type=jnp.float32)
    s_m = jnp.dot(src_oh, s.astype(jnp.bfloat16),
                  preferred_element_type=jnp.float32)
    log_s_m = jnp.log(s_m)
    tgt_oh = (jax.lax.broadcasted_iota(jnp.int32, (N, N), 1)
              == tgt_col).astype(jnp.float32)
    labels_m = jnp.dot(tgt_oh, tlabels2.reshape(N, 1),
                       preferred_element_type=jnp.float32)
    lab_oh = (jax.lax.broadcasted_iota(jnp.int32, (N, C1), 1)
              == labels_m.astype(jnp.int32))
    x_t = jnp.sum(jnp.where(lab_oh, rows_m, 0.0), axis=1, keepdims=True)
    x255_m = rows_m[:, num_classes:C1]
    valid = jnp.concatenate(
        [(src_col[1:, :] != src_col[:-1, :]).astype(jnp.float32),
         jnp.ones((1, 1), jnp.float32)], axis=0)
    ce_corr = jnp.sum(valid * (_focal(x_t - log_s_m)
                               - _focal(x255_m - log_s_m)))

    src_boxes_t = jnp.dot(boxes_t, matchf,
                          preferred_element_type=jnp.float32)
    src_cut = jnp.dot(cut_row, matchf,
                      preferred_element_type=jnp.float32)
    bbox_sum = jnp.sum(jnp.abs(src_boxes_t - tgt_boxes_t))
    cut_sum = jnp.sum(-(10.0 * tgt_cut_m * _log_sigmoid(src_cut)
                        + (1.0 - tgt_cut_m) * _log_sigmoid(-src_cut)))

    return (ce0_sum + ce_corr) / (B * Q) + 5.0 * bbox_sum / (B * N * 4) \
        + 2.0 * cut_sum / (B * N)


def _loss_body(logits_ref, boxes_ref, cut_ref, tboxes_ref, tlabels_ref,
               tcut_ref, src_ref, tgt_ref, out_ref, *, B, Q, C1, N):
    g = pl.program_id(0)
    part = 0.0
    for i in range(_BS):
        ba = g * _BS + i
        part = part + _batch_part(
            logits_ref[i], boxes_ref[i], cut_ref[pl.ds(ba, 1), :],
            tboxes_ref[i],
            tlabels_ref[pl.ds(ba, 1), :].astype(jnp.float32),
            tcut_ref[pl.ds(ba, 1), :].astype(jnp.float32),
            src_ref[pl.ds(ba, 1), :], tgt_ref[pl.ds(ba, 1), :],
            B=B, Q=Q, C1=C1, N=N)
    part = jnp.reshape(part, (1, 1))

    @pl.when(g == 0)
    def _():
        out_ref[:, :] = part

    @pl.when(g != 0)
    def _():
        out_ref[:, :] = out_ref[:, :] + part


@jax.jit
def kernel(pred_logits, pred_boxes, pred_cutting, target_boxes, target_labels,
           target_cutting, src_idx, tgt_idx):
    B, Q, C1 = pred_logits.shape
    N = src_idx.shape[1]
    boxes_t = jnp.transpose(pred_boxes, (0, 2, 1))
    tboxes_t = jnp.transpose(target_boxes, (0, 2, 1))

    out = pl.pallas_call(
        functools.partial(_loss_body, B=B, Q=Q, C1=C1, N=N),
        grid=(B // _BS,),
        in_specs=[
            pl.BlockSpec((_BS, Q, C1), lambda b: (b, 0, 0)),
            pl.BlockSpec((_BS, 4, Q), lambda b: (b, 0, 0)),
            pl.BlockSpec((B, Q), lambda b: (0, 0)),
            pl.BlockSpec((_BS, 4, N), lambda b: (b, 0, 0)),
            pl.BlockSpec((B, N), lambda b: (0, 0)),
            pl.BlockSpec((B, N), lambda b: (0, 0)),
            pl.BlockSpec((B, N), lambda b: (0, 0)),
            pl.BlockSpec((B, N), lambda b: (0, 0)),
        ],
        out_specs=pl.BlockSpec((1, 1), lambda b: (0, 0)),
        out_shape=jax.ShapeDtypeStruct((1, 1), jnp.float32),
    )(pred_logits, boxes_t, pred_cutting, tboxes_t,
      target_labels, target_cutting, src_idx, tgt_idx)
    return out.reshape(())


# speedup vs baseline: 1.0476x; 1.0476x over previous
import functools

import jax
import jax.numpy as jnp
from jax.experimental import pallas as pl

_BS = 8


def _log_sigmoid(x):
    return jnp.minimum(x, 0.0) - jnp.log1p(jnp.exp(-jnp.abs(x)))


def _focal4(logp):
    p = jnp.exp(logp)
    return (1.0 - p) ** 2 * logp


def _batch_part(logits, boxes_t, cut_row, tboxes_t, tlabels2, tcut2,
                src2, tgt2, *, B, Q, C1, N):
    num_classes = C1 - 1
    exp_x = jnp.exp(logits).astype(jnp.bfloat16)
    ones_c = jnp.ones((C1, 1), jnp.bfloat16)
    s = jnp.dot(exp_x, ones_c, preferred_element_type=jnp.float32)

    x255 = logits[:, num_classes:C1]
    ce0_sum = -0.25 * jnp.sum(_focal4(x255 - jnp.log(s)))

    tgt_ohT = (jax.lax.broadcasted_iota(jnp.int32, (N, N), 0)
               == tgt2).astype(jnp.float32)
    tgt_cut_m = jnp.dot(tcut2, tgt_ohT,
                        preferred_element_type=jnp.float32)
    tgt_boxes_t = jnp.dot(tboxes_t, tgt_ohT,
                          preferred_element_type=jnp.float32)

    matchf = (jax.lax.broadcasted_iota(jnp.int32, (Q, N), 0)
              == src2).astype(jnp.float32)

    src_col = src2.reshape(N, 1)
    tgt_col = tgt2.reshape(N, 1)
    src_oh = (jax.lax.broadcasted_iota(jnp.int32, (N, Q), 1)
              == src_col).astype(jnp.bfloat16)
    rows_m = jnp.dot(src_oh, logits.astype(jnp.bfloat16),
                     preferred_element_type=jnp.float32)
    s_m = jnp.dot(src_oh, s.astype(jnp.bfloat16),
                  preferred_element_type=jnp.float32)
    log_s_m = jnp.log(s_m)
    tgt_oh = (jax.lax.broadcasted_iota(jnp.int32, (N, N), 1)
              == tgt_col).astype(jnp.float32)
    labels_m = jnp.dot(tgt_oh, tlabels2.reshape(N, 1),
                       preferred_element_type=jnp.float32)
    lab_oh = (jax.lax.broadcasted_iota(jnp.int32, (N, C1), 1)
              == labels_m.astype(jnp.int32))
    x_t = jnp.sum(jnp.where(lab_oh, rows_m, 0.0), axis=1, keepdims=True)
    x255_m = rows_m[:, num_classes:C1]
    valid = jnp.concatenate(
        [(src_col[1:, :] != src_col[:-1, :]).astype(jnp.float32),
         jnp.ones((1, 1), jnp.float32)], axis=0)
    ce_corr = -0.25 * jnp.sum(valid * (_focal4(x_t - log_s_m)
                                       - _focal4(x255_m - log_s_m)))

    src_boxes_t = jnp.dot(boxes_t, matchf,
                          preferred_element_type=jnp.float32)
    src_cut = jnp.dot(cut_row, matchf,
                      preferred_element_type=jnp.float32)
    bbox_sum = jnp.sum(jnp.abs(src_boxes_t - tgt_boxes_t))
    cut_sum = jnp.sum(-(10.0 * tgt_cut_m * _log_sigmoid(src_cut)
                        + (1.0 - tgt_cut_m) * _log_sigmoid(-src_cut)))

    return (ce0_sum + ce_corr) / (B * Q) + 5.0 * bbox_sum / (B * N * 4) \
        + 2.0 * cut_sum / (B * N)


def _loss_body(logits_ref, boxes_ref, cut_ref, tboxes_ref, tlabels_ref,
               tcut_ref, src_ref, tgt_ref, out_ref, *, B, Q, C1, N):
    g = pl.program_id(0)
    part = 0.0
    for i in range(_BS):
        part = part + _batch_part(
            logits_ref[i], boxes_ref[i], cut_ref[pl.ds(i, 1), :],
            tboxes_ref[i],
            tlabels_ref[pl.ds(i, 1), :].astype(jnp.float32),
            tcut_ref[pl.ds(i, 1), :].astype(jnp.float32),
            src_ref[pl.ds(i, 1), :], tgt_ref[pl.ds(i, 1), :],
            B=B, Q=Q, C1=C1, N=N)
    part = jnp.reshape(part, (1, 1))

    @pl.when(g == 0)
    def _():
        out_ref[:, :] = part

    @pl.when(g != 0)
    def _():
        out_ref[:, :] = out_ref[:, :] + part


@jax.jit
def kernel(pred_logits, pred_boxes, pred_cutting, target_boxes, target_labels,
           target_cutting, src_idx, tgt_idx):
    B, Q, C1 = pred_logits.shape
    N = src_idx.shape[1]
    boxes_t = jnp.transpose(pred_boxes, (0, 2, 1))
    tboxes_t = jnp.transpose(target_boxes, (0, 2, 1))

    out = pl.pallas_call(
        functools.partial(_loss_body, B=B, Q=Q, C1=C1, N=N),
        grid=(B // _BS,),
        in_specs=[
            pl.BlockSpec((_BS, Q, C1), lambda b: (b, 0, 0)),
            pl.BlockSpec((_BS, 4, Q), lambda b: (b, 0, 0)),
            pl.BlockSpec((_BS, Q), lambda b: (b, 0)),
            pl.BlockSpec((_BS, 4, N), lambda b: (b, 0, 0)),
            pl.BlockSpec((_BS, N), lambda b: (b, 0)),
            pl.BlockSpec((_BS, N), lambda b: (b, 0)),
            pl.BlockSpec((_BS, N), lambda b: (b, 0)),
            pl.BlockSpec((_BS, N), lambda b: (b, 0)),
        ],
        out_specs=pl.BlockSpec((1, 1), lambda b: (0, 0)),
        out_shape=jax.ShapeDtypeStruct((1, 1), jnp.float32),
    )(pred_logits, boxes_t, pred_cutting, tboxes_t,
      target_labels, target_cutting, src_idx, tgt_idx)
    return out.reshape(())
